# baseline (speedup 1.0000x reference)
import functools

import jax
import jax.numpy as jnp
import numpy as np
from jax import lax
from jax.experimental import pallas as pl
from jax.experimental.pallas import tpu as pltpu
from jax.experimental.pallas import tpu_sc as plsc

T, N, E, B = 8, 10000, 320000, 64
FEAT, GCN, PROJ, KEYS, K = 2, 128, 256, 88, 3
EPS = 1e-5
ISQ = float(1.0 / np.sqrt(1.0 + EPS))

W16 = 16
GCNH = 64
EC = 125
KG = 8
GPT = E // (EC * KG)
NGRP = T * GPT
NC, NS = 2, 16
NW = NC * NS
GW = GPT // NW
WAVEN = 40
NWN = (GW * KG) // WAVEN
WAVE2 = 8
NW2 = (GW * KG) // WAVE2
NSPLIT = 5
NB = N // NSPLIT

ROWS_AB = T * N
RPS_AB = ROWS_AB // NS
RPS_C = N // NS

_MESH = plsc.VectorSubcoreMesh(core_axis_name="c", subcore_axis_name="s")
_SC_PARAMS = pltpu.CompilerParams(use_tc_tiling_on_sc=False)


def _worker_id():
    return lax.axis_index("s") * NC + lax.axis_index("c")


def _zero_vmem(zb, nrow, width):
    nv = width // 16

    def body(j, _):
        r = j // nv
        q = j % nv
        zb[r, pl.ds(q * 16, 16)] = jnp.zeros((16,), jnp.float32)
        return _

    lax.fori_loop(0, nrow * nv, body, None)


def _sc_deg_body(dstm3, out, idxb, ones_v, zb, spm, sems):
    c = lax.axis_index("c")
    s = lax.axis_index("s")
    wid = _worker_id()

    def ones_body(j, _):
        ones_v[j] = jnp.ones((16,), jnp.float32)
        return _

    lax.fori_loop(0, EC, ones_body, None)
    _zero_vmem(zb, 125, W16)
    for t0 in range(0, T, 2):
        for q in range(2 * RPS_C // 125):
            pltpu.sync_copy(zb, spm.at[pl.ds(s * 2 * RPS_C + q * 125, 125)])
        plsc.subcore_barrier()

        base = t0 * GPT + wid * GW

        def wave(i, _, base=base):
            g = base + (i // NWN) * GPT + (i % NWN) * (WAVEN // KG)
            pltpu.sync_copy(dstm3.at[pl.ds(g, WAVEN // KG)], idxb)
            descs = []
            for k in range(WAVEN):
                descs.append(
                    pltpu.async_copy(
                        ones_v, spm.at[idxb.at[k // KG].at[k % KG]], sems, add=True
                    )
                )
            for d in descs:
                d.wait()
            return _

        lax.fori_loop(0, 2 * NWN, wave, None)
        plsc.subcore_barrier()
        pltpu.sync_copy(
            spm.at[pl.ds(s * 2 * RPS_C, 2 * RPS_C)],
            out.at[c, pl.ds(t0 * N + s * 2 * RPS_C, 2 * RPS_C)],
        )
        plsc.subcore_barrier()


def _sc_deg(dstm3):
    return pl.kernel(
        _sc_deg_body,
        out_type=jax.ShapeDtypeStruct((NC, ROWS_AB, W16), jnp.float32),
        mesh=_MESH,
        compiler_params=_SC_PARAMS,
        scratch_types=[
            pltpu.VMEM((WAVEN // KG, KG, EC), jnp.int32),
            pltpu.VMEM((EC, W16), jnp.float32),
            pltpu.VMEM((125, W16), jnp.float32),
            pltpu.VMEM_SHARED((2 * N, W16), jnp.float32),
            pltpu.SemaphoreType.DMA,
        ],
    )(dstm3)


def _sc_agg1_body(w1, src3, dstm3, out, idxs, idxd, rows, zb, spm, gsem, ssem):
    c = lax.axis_index("c")
    s = lax.axis_index("s")
    wid = _worker_id()

    _zero_vmem(zb, 125, W16)
    for t0 in range(0, T, 2):
        for q in range(2 * RPS_C // 125):
            pltpu.sync_copy(zb, spm.at[pl.ds(s * 2 * RPS_C + q * 125, 125)])
        plsc.subcore_barrier()

        base = t0 * GPT + wid * GW

        def wave(i, _, base=base):
            g = base + (i // NWN) * GPT + (i % NWN) * (WAVEN // KG)
            pltpu.sync_copy(src3.at[pl.ds(g, WAVEN // KG)], idxs)
            pltpu.sync_copy(dstm3.at[pl.ds(g, WAVEN // KG)], idxd)
            gd = []
            for k in range(WAVEN):
                gd.append(
                    pltpu.async_copy(
                        w1.at[idxs.at[k // KG].at[k % KG]], rows.at[k], gsem
                    )
                )
            for d in gd:
                d.wait()
            sd = []
            for k in range(WAVEN):
                sd.append(
                    pltpu.async_copy(
                        rows.at[k], spm.at[idxd.at[k // KG].at[k % KG]], ssem, add=True
                    )
                )
            for d in sd:
                d.wait()
            return _

        lax.fori_loop(0, 2 * NWN, wave, None)
        plsc.subcore_barrier()
        pltpu.sync_copy(
            spm.at[pl.ds(s * 2 * RPS_C, 2 * RPS_C)],
            out.at[c, pl.ds(t0 * N + s * 2 * RPS_C, 2 * RPS_C)],
        )
        plsc.subcore_barrier()


def _sc_agg1(w1, src3, dstm3):
    return pl.kernel(
        _sc_agg1_body,
        out_type=jax.ShapeDtypeStruct((NC, ROWS_AB, W16), jnp.float32),
        mesh=_MESH,
        compiler_params=_SC_PARAMS,
        scratch_types=[
            pltpu.VMEM((WAVEN // KG, KG, EC), jnp.int32),
            pltpu.VMEM((WAVEN // KG, KG, EC), jnp.int32),
            pltpu.VMEM((WAVEN, EC, W16), jnp.float32),
            pltpu.VMEM((125, W16), jnp.float32),
            pltpu.VMEM_SHARED((2 * N, W16), jnp.float32),
            pltpu.SemaphoreType.DMA,
            pltpu.SemaphoreType.DMA,
        ],
    )(w1, src3, dstm3)


def _sc_agg2_body(w2, srcall, dst3, out, idxs, idxd, rows, zb, spm, gsem, ssem):
    c = lax.axis_index("c")
    s = lax.axis_index("s")
    wid = _worker_id()

    _zero_vmem(zb, 125, GCNH)

    def phase(p, _):
        t = p // 2
        h = p % 2
        for q in range(RPS_C // 125):
            pltpu.sync_copy(zb, spm.at[pl.ds(s * RPS_C + q * 125, 125)])
        plsc.subcore_barrier()

        gbase = t * GPT + wid * GW
        sbase = h * NGRP + gbase

        def wave(i, _):
            pltpu.sync_copy(srcall.at[sbase + i], idxs)
            pltpu.sync_copy(dst3.at[gbase + i], idxd)
            gd = []
            for k in range(WAVE2):
                gd.append(
                    pltpu.async_copy(w2.at[idxs.at[k]], rows.at[k], gsem)
                )
            for d in gd:
                d.wait()
            sd = []
            for k in range(WAVE2):
                sd.append(
                    pltpu.async_copy(rows.at[k], spm.at[idxd.at[k]], ssem, add=True)
                )
            for d in sd:
                d.wait()
            return _

        lax.fori_loop(0, NW2, wave, None)
        plsc.subcore_barrier()
        pltpu.sync_copy(
            spm.at[pl.ds(s * RPS_C, RPS_C)],
            out.at[c, pl.ds(h * ROWS_AB + t * N + s * RPS_C, RPS_C)],
        )
        plsc.subcore_barrier()
        return _

    lax.fori_loop(0, 2 * T, phase, None)


def _sc_agg2(w2, srcall, dst3):
    return pl.kernel(
        _sc_agg2_body,
        out_type=jax.ShapeDtypeStruct((NC, 2 * ROWS_AB, GCNH), jnp.float32),
        mesh=_MESH,
        compiler_params=_SC_PARAMS,
        scratch_types=[
            pltpu.VMEM((KG, EC), jnp.int32),
            pltpu.VMEM((KG, EC), jnp.int32),
            pltpu.VMEM((WAVE2, EC, GCNH), jnp.float32),
            pltpu.VMEM((125, GCNH), jnp.float32),
            pltpu.VMEM_SHARED((N, GCNH), jnp.float32),
            pltpu.SemaphoreType.DMA,
            pltpu.SemaphoreType.DMA,
        ],
    )(w2, srcall, dst3)


def _tc1_body(degp, x, w1):
    deg = degp[0, :, 0:1] + degp[1, :, 0:1] + 1.0
    dis = lax.rsqrt(deg)
    w1[:, 0:FEAT] = dis * x[0]
    w1[:, FEAT:W16] = jnp.zeros((NB, W16 - FEAT), jnp.float32)


def _tc1(degp, x):
    return pl.pallas_call(
        _tc1_body,
        grid=(T, NSPLIT),
        in_specs=[
            pl.BlockSpec((NC, NB, W16), lambda t, r: (0, t * NSPLIT + r, 0)),
            pl.BlockSpec((1, NB, FEAT), lambda t, r: (t, r, 0)),
        ],
        out_specs=pl.BlockSpec((NB, W16), lambda t, r: (t * NSPLIT + r, 0)),
        out_shape=jax.ShapeDtypeStruct((ROWS_AB, W16), jnp.float32),
    )(degp, x)


def _tc2_body(degp, agg1p, w1, W1p, b1, g1, be1, w2):
    deg = degp[0, :, 0:1] + degp[1, :, 0:1] + 1.0
    dis = lax.rsqrt(deg)
    a = agg1p[0] + agg1p[1] + w1[...]
    u = dis * a
    h = jnp.dot(u, W1p[...], preferred_element_type=jnp.float32) + b1[...]
    h = jnp.maximum(h * ISQ * g1[...] + be1[...], 0.0)
    w2f = dis * h
    w2[0] = w2f[:, 0:GCNH]
    w2[1] = w2f[:, GCNH:GCN]


def _tc2(degp, agg1p, w1, W1p, b1, g1, be1):
    cst = lambda shape: pl.BlockSpec(shape, lambda t, r: tuple(0 for _ in shape))
    return pl.pallas_call(
        _tc2_body,
        grid=(T, NSPLIT),
        in_specs=[
            pl.BlockSpec((NC, NB, W16), lambda t, r: (0, t * NSPLIT + r, 0)),
            pl.BlockSpec((NC, NB, W16), lambda t, r: (0, t * NSPLIT + r, 0)),
            pl.BlockSpec((NB, W16), lambda t, r: (t * NSPLIT + r, 0)),
            cst((W16, GCN)),
            cst((1, GCN)),
            cst((1, GCN)),
            cst((1, GCN)),
        ],
        out_specs=pl.BlockSpec((2, NB, GCNH), lambda t, r: (0, t * NSPLIT + r, 0)),
        out_shape=jax.ShapeDtypeStruct((2, ROWS_AB, GCNH), jnp.float32),
    )(degp, agg1p, w1, W1p, b1, g1, be1)


def _tc3_body(degp, agg2p, w2, bv, W2, b2, g2, be2, Wp, bp,
              cw1r, cg1, cb1, cw2r, cg2, cb2, Wh, bh, out, sacc, cacc, facc):
    t = pl.program_id(0)
    r = pl.program_id(1)
    deg = degp[0, :, 0:1] + degp[1, :, 0:1] + 1.0
    dis = lax.rsqrt(deg)
    w2c = jnp.concatenate([w2[0], w2[1]], axis=1)
    a2 = jnp.concatenate(
        [agg2p[0, 0] + agg2p[1, 0], agg2p[0, 1] + agg2p[1, 1]], axis=1)
    u = dis * (a2 + w2c)
    h = jnp.dot(u, W2[...], preferred_element_type=jnp.float32) + b2[...]
    h = jnp.maximum(h * ISQ * g2[...] + be2[...], 0.0)
    seg = bv[0, 0]
    ids = lax.broadcasted_iota(jnp.int32, (B, NB), 0)
    M = (ids == seg).astype(jnp.float32)
    part = jnp.dot(M, h, preferred_element_type=jnp.float32)
    pc = jnp.sum(M, axis=1, keepdims=True)

    @pl.when(r == 0)
    def _init():
        sacc[...] = part
        cacc[...] = pc

    @pl.when(r > 0)
    def _accum():
        sacc[...] += part
        cacc[...] += pc

    @pl.when(r == NSPLIT - 1)
    def _emit():
        pooled = sacc[...] / jnp.maximum(cacc[...], 1.0)
        facc[t] = jnp.dot(pooled, Wp[...], preferred_element_type=jnp.float32) + bp[...]

    @pl.when((t == T - 1) & (r == NSPLIT - 1))
    def _head():
        c1 = {}
        for tt in range(3, 6):
            acc = (
                facc[tt - 1] * cw1r[0:1, :]
                + facc[tt] * cw1r[1:2, :]
                + facc[tt + 1] * cw1r[2:3, :]
            )
            c1[tt] = jnp.maximum(acc * ISQ * cg1[...] + cb1[...], 0.0)
        acc2 = c1[3] * cw2r[0:1, :] + c1[4] * cw2r[1:2, :] + c1[5] * cw2r[2:3, :]
        center = jnp.maximum(acc2 * ISQ * cg2[...] + cb2[...], 0.0)
        out[...] = jnp.dot(center, Wh[...], preferred_element_type=jnp.float32) + bh[...]


def _tc3(degp, agg2p, w2, bv, W2, b2, g2, be2, Wp, bp,
         cw1r, cg1, cb1, cw2r, cg2, cb2, Wh, bh):
    cst = lambda shape: pl.BlockSpec(shape, lambda t, r: tuple(0 for _ in shape))
    return pl.pallas_call(
        _tc3_body,
        grid=(T, NSPLIT),
        in_specs=[
            pl.BlockSpec((NC, NB, W16), lambda t, r: (0, t * NSPLIT + r, 0)),
            pl.BlockSpec((NC, 2, NB, GCNH), lambda t, r: (0, 0, t * NSPLIT + r, 0)),
            pl.BlockSpec((2, NB, GCNH), lambda t, r: (0, t * NSPLIT + r, 0)),
            pl.BlockSpec((1, 1, 1, NB), lambda t, r: (t, r, 0, 0)),
            cst((GCN, GCN)),
            cst((1, GCN)),
            cst((1, GCN)),
            cst((1, GCN)),
            cst((GCN, PROJ)),
            cst((1, PROJ)),
            cst((K, PROJ)),
            cst((1, PROJ)),
            cst((1, PROJ)),
            cst((K, PROJ)),
            cst((1, PROJ)),
            cst((1, PROJ)),
            cst((PROJ, KEYS)),
            cst((1, KEYS)),
        ],
        out_specs=pl.BlockSpec((B, KEYS), lambda t, r: (0, 0)),
        out_shape=jax.ShapeDtypeStruct((B, KEYS), jnp.float32),
        scratch_shapes=[
            pltpu.VMEM((B, GCN), jnp.float32),
            pltpu.VMEM((B, 1), jnp.float32),
            pltpu.VMEM((T, B, PROJ), jnp.float32),
        ],
    )(degp, agg2p, w2, bv, W2, b2, g2, be2, Wp, bp,
      cw1r, cg1, cb1, cw2r, cg2, cb2, Wh, bh)


@jax.jit
def kernel(x, edge_index, batch_vec, W1, b1, W2, b2, g1, be1, g2, be2,
           Wp, bp, cw1, cg1, cb1, cw2, cg2, cb2, Wh, bh):
    bias = (jnp.arange(T, dtype=jnp.int32) * N)[:, None]
    src_f = edge_index[:, 0, :] + bias
    src3 = src_f.reshape(NGRP, KG, EC)
    src_all = jnp.concatenate([src3, src3 + ROWS_AB], axis=0)
    dst3 = edge_index[:, 1, :].reshape(NGRP, KG, EC)
    bias2 = ((jnp.arange(T, dtype=jnp.int32) % 2) * N)[:, None]
    dstm3 = (edge_index[:, 1, :] + bias2).reshape(NGRP, KG, EC)

    W1p = jnp.concatenate([W1, jnp.zeros((W16 - FEAT, GCN), jnp.float32)], axis=0)
    r = lambda v: v[None, :]
    cwa = jnp.transpose(cw1[:, 0, :], (1, 0))
    cwb = jnp.transpose(cw2[:, 0, :], (1, 0))

    degp = _sc_deg(dstm3)
    w1 = _tc1(degp, x)
    agg1p = _sc_agg1(w1, src3, dstm3)
    w2 = _tc2(degp, agg1p, w1, W1p, r(b1), r(g1), r(be1))
    w2f = w2.reshape(2 * ROWS_AB, GCNH)
    agg2p = _sc_agg2(w2f, src_all, dst3)
    agg2p = agg2p.reshape(NC, 2, ROWS_AB, GCNH)
    bv3 = batch_vec.reshape(T, NSPLIT, 1, NB)
    return _tc3(degp, agg2p, w2, bv3, W2, r(b2), r(g2), r(be2), Wp, r(bp),
                cwa, r(cg1), r(cb1), cwb, r(cg2), r(cb2), Wh, r(bh))


# speedup vs baseline: 1.0029x; 1.0029x over previous
import functools

import jax
import jax.numpy as jnp
import numpy as np
from jax import lax
from jax.experimental import pallas as pl
from jax.experimental.pallas import tpu as pltpu
from jax.experimental.pallas import tpu_sc as plsc

T, N, E, B = 8, 10000, 320000, 64
FEAT, GCN, PROJ, KEYS, K = 2, 128, 256, 88, 3
EPS = 1e-5
ISQ = float(1.0 / np.sqrt(1.0 + EPS))

W16 = 16
GCNH = 64
EC = 125
KG = 8
GPT = E // (EC * KG)
NGRP = T * GPT
NC, NS = 2, 16
NW = NC * NS
GW = GPT // NW
WAVEN = 40
NWN = (GW * KG) // WAVEN
WAVE2 = 8
NW2 = (GW * KG) // WAVE2
NSPLIT = 5
NB = N // NSPLIT

ROWS_AB = T * N
RPS_AB = ROWS_AB // NS
RPS_C = N // NS

_MESH = plsc.VectorSubcoreMesh(core_axis_name="c", subcore_axis_name="s")
_SC_PARAMS = pltpu.CompilerParams(use_tc_tiling_on_sc=False)


def _worker_id():
    return lax.axis_index("s") * NC + lax.axis_index("c")


def _zero_vmem(zb, nrow, width):
    nv = width // 16

    def body(j, _):
        r = j // nv
        q = j % nv
        zb[r, pl.ds(q * 16, 16)] = jnp.zeros((16,), jnp.float32)
        return _

    lax.fori_loop(0, nrow * nv, body, None)


def _sc_deg_body(dst3, out, idxb, ones_v, zb, spm, sems):
    c = lax.axis_index("c")
    s = lax.axis_index("s")
    wid = _worker_id()

    def ones_body(j, _):
        ones_v[j] = jnp.ones((16,), jnp.float32)
        return _

    lax.fori_loop(0, EC, ones_body, None)
    _zero_vmem(zb, 125, W16)
    for t in range(T):
        for q in range(RPS_C // 125):
            pltpu.sync_copy(zb, spm.at[pl.ds(s * RPS_C + q * 125, 125)])
        plsc.subcore_barrier()

        base = t * GPT + wid * GW

        def wave(i, _):
            pltpu.sync_copy(dst3.at[pl.ds(base + i * (WAVEN // KG), WAVEN // KG)], idxb)
            descs = []
            for k in range(WAVEN):
                descs.append(
                    pltpu.async_copy(
                        ones_v, spm.at[idxb.at[k // KG].at[k % KG]], sems, add=True
                    )
                )
            for d in descs:
                d.wait()
            return _

        lax.fori_loop(0, NWN, wave, None)
        plsc.subcore_barrier()
        pltpu.sync_copy(
            spm.at[pl.ds(s * RPS_C, RPS_C)],
            out.at[c, pl.ds(t * N + s * RPS_C, RPS_C)],
        )
        plsc.subcore_barrier()


def _sc_deg(dst3):
    return pl.kernel(
        _sc_deg_body,
        out_type=jax.ShapeDtypeStruct((NC, ROWS_AB, W16), jnp.float32),
        mesh=_MESH,
        compiler_params=_SC_PARAMS,
        scratch_types=[
            pltpu.VMEM((WAVEN // KG, KG, EC), jnp.int32),
            pltpu.VMEM((EC, W16), jnp.float32),
            pltpu.VMEM((125, W16), jnp.float32),
            pltpu.VMEM_SHARED((N, W16), jnp.float32),
            pltpu.SemaphoreType.DMA,
        ],
    )(dst3)


def _sc_agg1_body(w1, src3, dst3, out, idxs, idxd, rows, zb, spm, gsem, ssem):
    c = lax.axis_index("c")
    s = lax.axis_index("s")
    wid = _worker_id()

    _zero_vmem(zb, 125, W16)
    for t in range(T):
        for q in range(RPS_C // 125):
            pltpu.sync_copy(zb, spm.at[pl.ds(s * RPS_C + q * 125, 125)])
        plsc.subcore_barrier()

        base = t * GPT + wid * GW

        def wave(i, _):
            g = base + i * (WAVEN // KG)
            pltpu.sync_copy(src3.at[pl.ds(g, WAVEN // KG)], idxs)
            pltpu.sync_copy(dst3.at[pl.ds(g, WAVEN // KG)], idxd)
            gd = []
            for k in range(WAVEN):
                gd.append(
                    pltpu.async_copy(
                        w1.at[idxs.at[k // KG].at[k % KG]], rows.at[k], gsem
                    )
                )
            for d in gd:
                d.wait()
            sd = []
            for k in range(WAVEN):
                sd.append(
                    pltpu.async_copy(
                        rows.at[k], spm.at[idxd.at[k // KG].at[k % KG]], ssem, add=True
                    )
                )
            for d in sd:
                d.wait()
            return _

        lax.fori_loop(0, NWN, wave, None)
        plsc.subcore_barrier()
        pltpu.sync_copy(
            spm.at[pl.ds(s * RPS_C, RPS_C)],
            out.at[c, pl.ds(t * N + s * RPS_C, RPS_C)],
        )
        plsc.subcore_barrier()


def _sc_agg1(w1, src3, dst3):
    return pl.kernel(
        _sc_agg1_body,
        out_type=jax.ShapeDtypeStruct((NC, ROWS_AB, W16), jnp.float32),
        mesh=_MESH,
        compiler_params=_SC_PARAMS,
        scratch_types=[
            pltpu.VMEM((WAVEN // KG, KG, EC), jnp.int32),
            pltpu.VMEM((WAVEN // KG, KG, EC), jnp.int32),
            pltpu.VMEM((WAVEN, EC, W16), jnp.float32),
            pltpu.VMEM((125, W16), jnp.float32),
            pltpu.VMEM_SHARED((N, W16), jnp.float32),
            pltpu.SemaphoreType.DMA,
            pltpu.SemaphoreType.DMA,
        ],
    )(w1, src3, dst3)


def _sc_agg2_body(w2, srcall, dst3, out, idxs, idxd, rows, zb, spm, gsem, ssem):
    c = lax.axis_index("c")
    s = lax.axis_index("s")
    wid = _worker_id()

    _zero_vmem(zb, 125, GCNH)

    def phase(p, _):
        t = p // 2
        h = p % 2
        for q in range(RPS_C // 125):
            pltpu.sync_copy(zb, spm.at[pl.ds(s * RPS_C + q * 125, 125)])
        plsc.subcore_barrier()

        gbase = t * GPT + wid * GW
        sbase = h * NGRP + gbase

        def wave(i, _):
            pltpu.sync_copy(srcall.at[sbase + i], idxs)
            pltpu.sync_copy(dst3.at[gbase + i], idxd)
            gd = []
            for k in range(WAVE2):
                gd.append(
                    pltpu.async_copy(w2.at[idxs.at[k]], rows.at[k], gsem)
                )
            for d in gd:
                d.wait()
            sd = []
            for k in range(WAVE2):
                sd.append(
                    pltpu.async_copy(rows.at[k], spm.at[idxd.at[k]], ssem, add=True)
                )
            for d in sd:
                d.wait()
            return _

        lax.fori_loop(0, NW2, wave, None)
        plsc.subcore_barrier()
        pltpu.sync_copy(
            spm.at[pl.ds(s * RPS_C, RPS_C)],
            out.at[c, pl.ds(h * ROWS_AB + t * N + s * RPS_C, RPS_C)],
        )
        plsc.subcore_barrier()
        return _

    lax.fori_loop(0, 2 * T, phase, None)


def _sc_agg2(w2, srcall, dst3):
    return pl.kernel(
        _sc_agg2_body,
        out_type=jax.ShapeDtypeStruct((NC, 2 * ROWS_AB, GCNH), jnp.float32),
        mesh=_MESH,
        compiler_params=_SC_PARAMS,
        scratch_types=[
            pltpu.VMEM((KG, EC), jnp.int32),
            pltpu.VMEM((KG, EC), jnp.int32),
            pltpu.VMEM((WAVE2, EC, GCNH), jnp.float32),
            pltpu.VMEM((125, GCNH), jnp.float32),
            pltpu.VMEM_SHARED((N, GCNH), jnp.float32),
            pltpu.SemaphoreType.DMA,
            pltpu.SemaphoreType.DMA,
        ],
    )(w2, srcall, dst3)


def _tc1_body(degp, x, w1):
    deg = degp[0, :, 0:1] + degp[1, :, 0:1] + 1.0
    dis = lax.rsqrt(deg)
    w1[:, 0:FEAT] = dis * x[0]
    w1[:, FEAT:W16] = jnp.zeros((NB, W16 - FEAT), jnp.float32)


def _tc1(degp, x):
    return pl.pallas_call(
        _tc1_body,
        grid=(T, NSPLIT),
        in_specs=[
            pl.BlockSpec((NC, NB, W16), lambda t, r: (0, t * NSPLIT + r, 0)),
            pl.BlockSpec((1, NB, FEAT), lambda t, r: (t, r, 0)),
        ],
        out_specs=pl.BlockSpec((NB, W16), lambda t, r: (t * NSPLIT + r, 0)),
        out_shape=jax.ShapeDtypeStruct((ROWS_AB, W16), jnp.float32),
    )(degp, x)


def _tc2_body(degp, agg1p, w1, W1p, b1, g1, be1, w2):
    deg = degp[0, :, 0:1] + degp[1, :, 0:1] + 1.0
    dis = lax.rsqrt(deg)
    a = agg1p[0] + agg1p[1] + w1[...]
    u = dis * a
    h = jnp.dot(u, W1p[...], preferred_element_type=jnp.float32) + b1[...]
    h = jnp.maximum(h * ISQ * g1[...] + be1[...], 0.0)
    w2f = dis * h
    w2[0] = w2f[:, 0:GCNH]
    w2[1] = w2f[:, GCNH:GCN]


def _tc2(degp, agg1p, w1, W1p, b1, g1, be1):
    cst = lambda shape: pl.BlockSpec(shape, lambda t, r: tuple(0 for _ in shape))
    return pl.pallas_call(
        _tc2_body,
        grid=(T, NSPLIT),
        in_specs=[
            pl.BlockSpec((NC, NB, W16), lambda t, r: (0, t * NSPLIT + r, 0)),
            pl.BlockSpec((NC, NB, W16), lambda t, r: (0, t * NSPLIT + r, 0)),
            pl.BlockSpec((NB, W16), lambda t, r: (t * NSPLIT + r, 0)),
            cst((W16, GCN)),
            cst((1, GCN)),
            cst((1, GCN)),
            cst((1, GCN)),
        ],
        out_specs=pl.BlockSpec((2, NB, GCNH), lambda t, r: (0, t * NSPLIT + r, 0)),
        out_shape=jax.ShapeDtypeStruct((2, ROWS_AB, GCNH), jnp.float32),
    )(degp, agg1p, w1, W1p, b1, g1, be1)


def _tc3_body(degp, agg2p, w2, bv, W2, b2, g2, be2, Wp, bp,
              cw1r, cg1, cb1, cw2r, cg2, cb2, Wh, bh, out, sacc, cacc, facc):
    t = pl.program_id(0)
    r = pl.program_id(1)
    deg = degp[0, :, 0:1] + degp[1, :, 0:1] + 1.0
    dis = lax.rsqrt(deg)
    w2c = jnp.concatenate([w2[0], w2[1]], axis=1)
    a2 = jnp.concatenate(
        [agg2p[0, 0] + agg2p[1, 0], agg2p[0, 1] + agg2p[1, 1]], axis=1)
    u = dis * (a2 + w2c)
    h = jnp.dot(u, W2[...], preferred_element_type=jnp.float32) + b2[...]
    h = jnp.maximum(h * ISQ * g2[...] + be2[...], 0.0)
    seg = bv[0, 0]
    ids = lax.broadcasted_iota(jnp.int32, (B, NB), 0)
    M = (ids == seg).astype(jnp.float32)
    part = jnp.dot(M, h, preferred_element_type=jnp.float32)
    pc = jnp.sum(M, axis=1, keepdims=True)

    @pl.when(r == 0)
    def _init():
        sacc[...] = part
        cacc[...] = pc

    @pl.when(r > 0)
    def _accum():
        sacc[...] += part
        cacc[...] += pc

    @pl.when(r == NSPLIT - 1)
    def _emit():
        pooled = sacc[...] / jnp.maximum(cacc[...], 1.0)
        facc[t] = jnp.dot(pooled, Wp[...], preferred_element_type=jnp.float32) + bp[...]

    @pl.when((t == T - 1) & (r == NSPLIT - 1))
    def _head():
        c1 = {}
        for tt in range(3, 6):
            acc = (
                facc[tt - 1] * cw1r[0:1, :]
                + facc[tt] * cw1r[1:2, :]
                + facc[tt + 1] * cw1r[2:3, :]
            )
            c1[tt] = jnp.maximum(acc * ISQ * cg1[...] + cb1[...], 0.0)
        acc2 = c1[3] * cw2r[0:1, :] + c1[4] * cw2r[1:2, :] + c1[5] * cw2r[2:3, :]
        center = jnp.maximum(acc2 * ISQ * cg2[...] + cb2[...], 0.0)
        out[...] = jnp.dot(center, Wh[...], preferred_element_type=jnp.float32) + bh[...]


def _tc3(degp, agg2p, w2, bv, W2, b2, g2, be2, Wp, bp,
         cw1r, cg1, cb1, cw2r, cg2, cb2, Wh, bh):
    cst = lambda shape: pl.BlockSpec(shape, lambda t, r: tuple(0 for _ in shape))
    return pl.pallas_call(
        _tc3_body,
        grid=(T, NSPLIT),
        in_specs=[
            pl.BlockSpec((NC, NB, W16), lambda t, r: (0, t * NSPLIT + r, 0)),
            pl.BlockSpec((NC, 2, NB, GCNH), lambda t, r: (0, 0, t * NSPLIT + r, 0)),
            pl.BlockSpec((2, NB, GCNH), lambda t, r: (0, t * NSPLIT + r, 0)),
            pl.BlockSpec((1, 1, 1, NB), lambda t, r: (t, r, 0, 0)),
            cst((GCN, GCN)),
            cst((1, GCN)),
            cst((1, GCN)),
            cst((1, GCN)),
            cst((GCN, PROJ)),
            cst((1, PROJ)),
            cst((K, PROJ)),
            cst((1, PROJ)),
            cst((1, PROJ)),
            cst((K, PROJ)),
            cst((1, PROJ)),
            cst((1, PROJ)),
            cst((PROJ, KEYS)),
            cst((1, KEYS)),
        ],
        out_specs=pl.BlockSpec((B, KEYS), lambda t, r: (0, 0)),
        out_shape=jax.ShapeDtypeStruct((B, KEYS), jnp.float32),
        scratch_shapes=[
            pltpu.VMEM((B, GCN), jnp.float32),
            pltpu.VMEM((B, 1), jnp.float32),
            pltpu.VMEM((T, B, PROJ), jnp.float32),
        ],
    )(degp, agg2p, w2, bv, W2, b2, g2, be2, Wp, bp,
      cw1r, cg1, cb1, cw2r, cg2, cb2, Wh, bh)


@jax.jit
def kernel(x, edge_index, batch_vec, W1, b1, W2, b2, g1, be1, g2, be2,
           Wp, bp, cw1, cg1, cb1, cw2, cg2, cb2, Wh, bh):
    bias = (jnp.arange(T, dtype=jnp.int32) * N)[:, None]
    src_f = edge_index[:, 0, :] + bias
    src3 = src_f.reshape(NGRP, KG, EC)
    src_all = jnp.concatenate([src3, src3 + ROWS_AB], axis=0)
    dst3 = edge_index[:, 1, :].reshape(NGRP, KG, EC)

    W1p = jnp.concatenate([W1, jnp.zeros((W16 - FEAT, GCN), jnp.float32)], axis=0)
    r = lambda v: v[None, :]
    cwa = jnp.transpose(cw1[:, 0, :], (1, 0))
    cwb = jnp.transpose(cw2[:, 0, :], (1, 0))

    degp = _sc_deg(dst3)
    w1 = _tc1(degp, x)
    agg1p = _sc_agg1(w1, src3, dst3)
    w2 = _tc2(degp, agg1p, w1, W1p, r(b1), r(g1), r(be1))
    w2f = w2.reshape(2 * ROWS_AB, GCNH)
    agg2p = _sc_agg2(w2f, src_all, dst3)
    agg2p = agg2p.reshape(NC, 2, ROWS_AB, GCNH)
    bv3 = batch_vec.reshape(T, NSPLIT, 1, NB)
    return _tc3(degp, agg2p, w2, bv3, W2, r(b2), r(g2), r(be2), Wp, r(bp),
                cwa, r(cg1), r(cb1), cwb, r(cg2), r(cb2), Wh, r(bh))
